# Initial kernel scaffold; baseline (speedup 1.0000x reference)
import jax
import jax.numpy as jnp
from jax.experimental import pallas as pl


def kernel(imgs, depths, fmap0, fmap1):
    raise NotImplementedError("write your pallas kernel here")


# speedup vs baseline: 50.6241x; 50.6241x over previous
import functools

import jax
import jax.numpy as jnp
from jax.experimental import pallas as pl
from jax.experimental.pallas import tpu as pltpu

_LAMBDA = 0.1
_D = 64
_H = 224
_W = 224
_TILE = 32
_NT = _H // _TILE
_S0, _C0 = 56, 96
_S1, _C1 = 28, 192
_HIGH = jax.lax.Precision.HIGHEST


def _bin_index(depth):
    step = jnp.float32(15.625)
    b0f = jnp.floor(depth * jnp.float32(1.0 / 15.625))
    up = ((b0f + 1.0) * step <= depth).astype(jnp.int32)
    dn = (b0f * step > depth).astype(jnp.int32)
    b = b0f.astype(jnp.int32) + up - dn
    return jnp.clip(b, 0, _D - 1)


def _body(depth_ref, f0_ref, f1_ref, u0_ref, u1_ref, ut0_ref, ut1_ref, out_ref,
          wt0_ref, wt1_ref, ht0_ref, ht1_ref, sums0_ref, sums1_ref, cnt_ref):
    b = pl.program_id(0)
    t = pl.program_id(1)

    @pl.when((b == 0) & (t == 0))
    def _init():
        sums0_ref[...] = jnp.zeros_like(sums0_ref)
        sums1_ref[...] = jnp.zeros_like(sums1_ref)
        cnt_ref[...] = jnp.zeros_like(cnt_ref)

    bins = _bin_index(depth_ref[0])

    iota_d = jax.lax.broadcasted_iota(jnp.int32, (_D, _TILE, _W), 0)
    oh = (bins[None, :, :] == iota_d).astype(jnp.float32)
    cnt_ref[...] = cnt_ref[...] + jnp.sum(oh, axis=(1, 2))[:, None]
    oh2 = oh.reshape(_D * _TILE, _W)

    w0_t = jax.lax.dot_general(oh2, u0_ref[...], (((1,), (0,)), ((), ())),
                               precision=_HIGH)
    wt0_ref[...] = w0_t.reshape(_D, _TILE, _S0)
    ht0_ref[...] = jax.lax.dot_general(ut0_ref[...], f0_ref[0],
                                       (((1,), (0,)), ((), ())),
                                       precision=_HIGH)
    sums0_ref[...] = sums0_ref[...] + jax.lax.dot_general(
        wt0_ref[...].reshape(_D, _TILE * _S0),
        ht0_ref[...].reshape(_TILE * _S0, _C0),
        (((1,), (0,)), ((), ())), precision=_HIGH)

    w1_t = jax.lax.dot_general(oh2, u1_ref[...], (((1,), (0,)), ((), ())),
                               precision=_HIGH)
    wt1_ref[...] = w1_t.reshape(_D, _TILE, _S1)
    ht1_ref[...] = jax.lax.dot_general(ut1_ref[...], f1_ref[0],
                                       (((1,), (0,)), ((), ())),
                                       precision=_HIGH)
    sums1_ref[...] = sums1_ref[...] + jax.lax.dot_general(
        wt1_ref[...].reshape(_D, _TILE * _S1),
        ht1_ref[...].reshape(_TILE * _S1, _C1),
        (((1,), (0,)), ((), ())), precision=_HIGH)

    @pl.when((b == pl.num_programs(0) - 1) & (t == pl.num_programs(1) - 1))
    def _epilogue():
        cnt = cnt_ref[...]
        nonzero = cnt > 0.0
        denom = jnp.maximum(cnt, 1.0)

        def layer_term(sums, c):
            means = jnp.where(nonzero, sums / denom, 0.0)
            k = jax.lax.broadcasted_iota(jnp.int32, (_D, c), 1)
            d = jax.lax.broadcasted_iota(jnp.int32, (_D, c), 0)
            d_k = jnp.clip((k * 64) // c, 0, _D - 1)
            mask = (d == d_k).astype(jnp.float32)
            r_dk = jnp.sum(means * mask, axis=0, keepdims=True)
            sum_all = jnp.sum(means, axis=0, keepdims=True)
            r_rest = (sum_all - r_dk) / jnp.float32(_D - 1)
            aa = jnp.abs(r_dk)
            ab = jnp.abs(r_rest)
            s_k = (aa - ab) / (aa + ab + jnp.float32(1e-6))
            return jnp.sum(s_k) / jnp.float32(c)

        total = layer_term(sums0_ref[...], _C0) + layer_term(sums1_ref[...], _C1)
        val = jnp.float32(-_LAMBDA) * (total / jnp.float32(2.0))
        out_ref[...] = val.reshape(1, 1)


@functools.partial(jax.jit, static_argnums=())
def _run(depths_s, f0r, f1r, u0, u1):
    out = pl.pallas_call(
        _body,
        grid=(4, _NT),
        in_specs=[
            pl.BlockSpec((1, _TILE, _W), lambda b, t: (b, t, 0)),
            pl.BlockSpec((1, _S0, _S0, _C0), lambda b, t: (b, 0, 0, 0)),
            pl.BlockSpec((1, _S1, _S1, _C1), lambda b, t: (b, 0, 0, 0)),
            pl.BlockSpec((_H, _S0), lambda b, t: (0, 0)),
            pl.BlockSpec((_H, _S1), lambda b, t: (0, 0)),
            pl.BlockSpec((_TILE, _S0), lambda b, t: (t, 0)),
            pl.BlockSpec((_TILE, _S1), lambda b, t: (t, 0)),
        ],
        out_specs=pl.BlockSpec((1, 1), lambda b, t: (0, 0)),
        out_shape=jax.ShapeDtypeStruct((1, 1), jnp.float32),
        scratch_shapes=[
            pltpu.VMEM((_D, _TILE, _S0), jnp.float32),
            pltpu.VMEM((_D, _TILE, _S1), jnp.float32),
            pltpu.VMEM((_TILE, _S0, _C0), jnp.float32),
            pltpu.VMEM((_TILE, _S1, _C1), jnp.float32),
            pltpu.VMEM((_D, _C0), jnp.float32),
            pltpu.VMEM((_D, _C1), jnp.float32),
            pltpu.VMEM((_D, 1), jnp.float32),
        ],
        compiler_params=pltpu.CompilerParams(
            dimension_semantics=("arbitrary", "arbitrary"),
        ),
    )(depths_s, f0r, f1r, u0, u1, u0, u1)
    return out.reshape(())


def kernel(imgs, depths, fmap0, fmap1):
    del imgs
    depths_s = depths[:, 0, :, :]
    u0 = jax.image.resize(jnp.eye(_S0, dtype=jnp.float32), (_H, _S0), "bilinear")
    u1 = jax.image.resize(jnp.eye(_S1, dtype=jnp.float32), (_H, _S1), "bilinear")
    f0r = jnp.transpose(fmap0, (0, 2, 3, 1))
    f1r = jnp.transpose(fmap1, (0, 2, 3, 1))
    return _run(depths_s, f0r, f1r, u0, u1)


# speedup vs baseline: 131.4820x; 2.5972x over previous
import functools

import jax
import jax.numpy as jnp
from jax.experimental import pallas as pl
from jax.experimental.pallas import tpu as pltpu

_LAMBDA = 0.1
_D = 64
_H = 224
_W = 224
_TILE = 56
_NT = _H // _TILE
_S0, _C0 = 56, 96
_S1, _C1 = 28, 192
_HIGH = jax.lax.Precision.HIGHEST


def _bin_index(depth):
    step = jnp.float32(15.625)
    b0f = jnp.floor(depth * jnp.float32(1.0 / 15.625))
    up = ((b0f + 1.0) * step <= depth).astype(jnp.int32)
    dn = (b0f * step > depth).astype(jnp.int32)
    b = b0f.astype(jnp.int32) + up - dn
    return jnp.clip(b, 0, _D - 1)


def _body(depth_ref, f0_ref, f1_ref, u0_ref, u1_ref, ut0_ref, ut1_ref, out_ref,
          wt0_ref, wt1_ref, h0hi_ref, h0lo_ref, h1hi_ref, h1lo_ref,
          sums0_ref, sums1_ref, cnt_ref):
    b = pl.program_id(0)
    t = pl.program_id(1)

    @pl.when((b == 0) & (t == 0))
    def _init():
        sums0_ref[...] = jnp.zeros_like(sums0_ref)
        sums1_ref[...] = jnp.zeros_like(sums1_ref)
        cnt_ref[...] = jnp.zeros_like(cnt_ref)

    bins = _bin_index(depth_ref[0])

    iota_d = jax.lax.broadcasted_iota(jnp.int32, (_D, _TILE, _W), 0)
    oh = (bins[None, :, :] == iota_d).astype(jnp.bfloat16)
    oh2 = oh.reshape(_D * _TILE, _W)

    def layer(u_ref, ut_ref, f_ref, wt_ref, hhi_ref, hlo_ref, s, c):
        w_t = jax.lax.dot_general(oh2, u_ref[...], (((1,), (0,)), ((), ())),
                                  preferred_element_type=jnp.float32)
        wt_ref[...] = w_t.reshape(_D, _TILE, s).astype(jnp.bfloat16)
        ht = jax.lax.dot_general(ut_ref[...], f_ref[0],
                                 (((1,), (0,)), ((), ())),
                                 precision=_HIGH)
        hhi = ht.astype(jnp.bfloat16)
        hhi_ref[...] = hhi
        hlo_ref[...] = (ht - hhi.astype(jnp.float32)).astype(jnp.bfloat16)
        w2 = wt_ref[...].reshape(_D, _TILE * s)
        return (jax.lax.dot_general(w2, hhi_ref[...].reshape(_TILE * s, c),
                                    (((1,), (0,)), ((), ())),
                                    preferred_element_type=jnp.float32)
                + jax.lax.dot_general(w2, hlo_ref[...].reshape(_TILE * s, c),
                                      (((1,), (0,)), ((), ())),
                                      preferred_element_type=jnp.float32))

    sums0_ref[...] = sums0_ref[...] + layer(
        u0_ref, ut0_ref, f0_ref, wt0_ref, h0hi_ref, h0lo_ref, _S0, _C0)
    sums1_ref[...] = sums1_ref[...] + layer(
        u1_ref, ut1_ref, f1_ref, wt1_ref, h1hi_ref, h1lo_ref, _S1, _C1)
    cnt_ref[...] = cnt_ref[...] + jnp.sum(
        wt0_ref[...].astype(jnp.float32), axis=(1, 2))[:, None]

    @pl.when((b == pl.num_programs(0) - 1) & (t == pl.num_programs(1) - 1))
    def _epilogue():
        cnt = cnt_ref[...]
        nonzero = cnt > 0.5
        denom = jnp.maximum(cnt, 1.0)

        def layer_term(sums, c):
            means = jnp.where(nonzero, sums / denom, 0.0)
            k = jax.lax.broadcasted_iota(jnp.int32, (_D, c), 1)
            d = jax.lax.broadcasted_iota(jnp.int32, (_D, c), 0)
            d_k = jnp.clip((k * 64) // c, 0, _D - 1)
            mask = (d == d_k).astype(jnp.float32)
            r_dk = jnp.sum(means * mask, axis=0, keepdims=True)
            sum_all = jnp.sum(means, axis=0, keepdims=True)
            r_rest = (sum_all - r_dk) / jnp.float32(_D - 1)
            aa = jnp.abs(r_dk)
            ab = jnp.abs(r_rest)
            s_k = (aa - ab) / (aa + ab + jnp.float32(1e-6))
            return jnp.sum(s_k) / jnp.float32(c)

        total = layer_term(sums0_ref[...], _C0) + layer_term(sums1_ref[...], _C1)
        val = jnp.float32(-_LAMBDA) * (total / jnp.float32(2.0))
        out_ref[...] = val.reshape(1, 1)


@functools.partial(jax.jit, static_argnums=())
def _run(depths_s, f0r, f1r, u0b, u1b, u0, u1):
    out = pl.pallas_call(
        _body,
        grid=(4, _NT),
        in_specs=[
            pl.BlockSpec((1, _TILE, _W), lambda b, t: (b, t, 0)),
            pl.BlockSpec((1, _S0, _S0, _C0), lambda b, t: (b, 0, 0, 0)),
            pl.BlockSpec((1, _S1, _S1, _C1), lambda b, t: (b, 0, 0, 0)),
            pl.BlockSpec((_H, _S0), lambda b, t: (0, 0)),
            pl.BlockSpec((_H, _S1), lambda b, t: (0, 0)),
            pl.BlockSpec((_TILE, _S0), lambda b, t: (t, 0)),
            pl.BlockSpec((_TILE, _S1), lambda b, t: (t, 0)),
        ],
        out_specs=pl.BlockSpec((1, 1), lambda b, t: (0, 0)),
        out_shape=jax.ShapeDtypeStruct((1, 1), jnp.float32),
        scratch_shapes=[
            pltpu.VMEM((_D, _TILE, _S0), jnp.bfloat16),
            pltpu.VMEM((_D, _TILE, _S1), jnp.bfloat16),
            pltpu.VMEM((_TILE, _S0, _C0), jnp.bfloat16),
            pltpu.VMEM((_TILE, _S0, _C0), jnp.bfloat16),
            pltpu.VMEM((_TILE, _S1, _C1), jnp.bfloat16),
            pltpu.VMEM((_TILE, _S1, _C1), jnp.bfloat16),
            pltpu.VMEM((_D, _C0), jnp.float32),
            pltpu.VMEM((_D, _C1), jnp.float32),
            pltpu.VMEM((_D, 1), jnp.float32),
        ],
        compiler_params=pltpu.CompilerParams(
            dimension_semantics=("arbitrary", "arbitrary"),
        ),
    )(depths_s, f0r, f1r, u0b, u1b, u0, u1)
    return out.reshape(())


def kernel(imgs, depths, fmap0, fmap1):
    del imgs
    depths_s = depths[:, 0, :, :]
    u0 = jax.image.resize(jnp.eye(_S0, dtype=jnp.float32), (_H, _S0), "bilinear")
    u1 = jax.image.resize(jnp.eye(_S1, dtype=jnp.float32), (_H, _S1), "bilinear")
    u0b = u0.astype(jnp.bfloat16)
    u1b = u1.astype(jnp.bfloat16)
    f0r = jnp.transpose(fmap0, (0, 2, 3, 1))
    f1r = jnp.transpose(fmap1, (0, 2, 3, 1))
    return _run(depths_s, f0r, f1r, u0b, u1b, u0, u1)
